# baseline (speedup 1.0000x reference)
import functools

import jax
import jax.numpy as jnp
from jax.experimental import pallas as pl
from jax.experimental.pallas import tpu as pltpu

M = 2 * 77
K = 49408
N = 768
KB = 4096
NSTEPS = -(-K // KB)


def _body(a_ref, b_ref, p_ref, o_ref):
    k = pl.program_id(0)

    def full_dot():
        return jnp.dot(a_ref[...], b_ref[...],
                       preferred_element_type=jnp.float32)

    def masked_dot():
        valid = K - (NSTEPS - 1) * KB
        a = a_ref[...]
        b = b_ref[...]
        a = jnp.where(
            jax.lax.broadcasted_iota(jnp.int32, a.shape, 1) < valid, a, 0.0)
        b = jnp.where(
            jax.lax.broadcasted_iota(jnp.int32, b.shape, 0) < valid, b, 0.0)
        return jnp.dot(a, b, preferred_element_type=jnp.float32)

    partial = jax.lax.cond(k == NSTEPS - 1, masked_dot, full_dot)

    @pl.when(k == 0)
    def _init():
        p = p_ref[...]
        o_ref[...] = partial + jnp.concatenate([p, p], axis=0)

    @pl.when(k > 0)
    def _acc():
        o_ref[...] += partial


@jax.jit
def kernel(input_ids, token_weight, position_weight):
    batch, seq, _ = input_ids.shape
    a2d = input_ids.reshape(batch * seq, K)
    out2d = pl.pallas_call(
        _body,
        grid=(NSTEPS,),
        in_specs=[
            pl.BlockSpec((M, KB), lambda k: (0, k)),
            pl.BlockSpec((KB, N), lambda k: (k, 0)),
            pl.BlockSpec((seq, N), lambda k: (0, 0)),
        ],
        out_specs=pl.BlockSpec((M, N), lambda k: (0, 0)),
        out_shape=jax.ShapeDtypeStruct((M, N), jnp.float32),
        compiler_params=pltpu.CompilerParams(
            dimension_semantics=("arbitrary",)),
    )(a2d, token_weight, position_weight)
    return out2d.reshape(batch, seq, N)


# speedup vs baseline: 1.0007x; 1.0007x over previous
import functools

import jax
import jax.numpy as jnp
from jax.experimental import pallas as pl
from jax.experimental.pallas import tpu as pltpu

M = 2 * 77
K = 49408
N = 768
KB = 4096
NSTEPS = -(-K // KB)


def _body(a_ref, b_ref, p_ref, o_ref):
    k = pl.program_id(0)

    def full_dot():
        return jnp.dot(a_ref[...].astype(jnp.bfloat16),
                       b_ref[...].astype(jnp.bfloat16),
                       preferred_element_type=jnp.float32)

    def masked_dot():
        valid = K - (NSTEPS - 1) * KB
        a = a_ref[...]
        b = b_ref[...]
        a = jnp.where(
            jax.lax.broadcasted_iota(jnp.int32, a.shape, 1) < valid, a, 0.0)
        b = jnp.where(
            jax.lax.broadcasted_iota(jnp.int32, b.shape, 0) < valid, b, 0.0)
        return jnp.dot(a.astype(jnp.bfloat16), b.astype(jnp.bfloat16),
                       preferred_element_type=jnp.float32)

    partial = jax.lax.cond(k == NSTEPS - 1, masked_dot, full_dot)

    @pl.when(k == 0)
    def _init():
        p = p_ref[...]
        o_ref[...] = partial + jnp.concatenate([p, p], axis=0)

    @pl.when(k > 0)
    def _acc():
        o_ref[...] += partial


@jax.jit
def kernel(input_ids, token_weight, position_weight):
    batch, seq, _ = input_ids.shape
    a2d = input_ids.reshape(batch * seq, K)
    out2d = pl.pallas_call(
        _body,
        grid=(NSTEPS,),
        in_specs=[
            pl.BlockSpec((M, KB), lambda k: (0, k)),
            pl.BlockSpec((KB, N), lambda k: (k, 0)),
            pl.BlockSpec((seq, N), lambda k: (0, 0)),
        ],
        out_specs=pl.BlockSpec((M, N), lambda k: (0, 0)),
        out_shape=jax.ShapeDtypeStruct((M, N), jnp.float32),
        compiler_params=pltpu.CompilerParams(
            dimension_semantics=("arbitrary",)),
    )(a2d, token_weight, position_weight)
    return out2d.reshape(batch, seq, N)


# speedup vs baseline: 1.0196x; 1.0189x over previous
import functools

import jax
import jax.numpy as jnp
from jax.experimental import pallas as pl
from jax.experimental.pallas import tpu as pltpu

M = 2 * 77
K = 49408
N = 768
KB = 2048
NSTEPS = -(-K // KB)


def _body(a_ref, b_ref, p_ref, o_ref):
    k = pl.program_id(0)

    def full_dot():
        return jnp.dot(a_ref[...].astype(jnp.bfloat16),
                       b_ref[...].astype(jnp.bfloat16),
                       preferred_element_type=jnp.float32)

    def masked_dot():
        valid = K - (NSTEPS - 1) * KB
        a = a_ref[...]
        b = b_ref[...]
        a = jnp.where(
            jax.lax.broadcasted_iota(jnp.int32, a.shape, 1) < valid, a, 0.0)
        b = jnp.where(
            jax.lax.broadcasted_iota(jnp.int32, b.shape, 0) < valid, b, 0.0)
        return jnp.dot(a.astype(jnp.bfloat16), b.astype(jnp.bfloat16),
                       preferred_element_type=jnp.float32)

    partial = jax.lax.cond(k == NSTEPS - 1, masked_dot, full_dot)

    @pl.when(k == 0)
    def _init():
        p = p_ref[...]
        o_ref[...] = partial + jnp.concatenate([p, p], axis=0)

    @pl.when(k > 0)
    def _acc():
        o_ref[...] += partial


@jax.jit
def kernel(input_ids, token_weight, position_weight):
    batch, seq, _ = input_ids.shape
    a2d = input_ids.reshape(batch * seq, K)
    out2d = pl.pallas_call(
        _body,
        grid=(NSTEPS,),
        in_specs=[
            pl.BlockSpec((M, KB), lambda k: (0, k)),
            pl.BlockSpec((KB, N), lambda k: (k, 0)),
            pl.BlockSpec((seq, N), lambda k: (0, 0)),
        ],
        out_specs=pl.BlockSpec((M, N), lambda k: (0, 0)),
        out_shape=jax.ShapeDtypeStruct((M, N), jnp.float32),
        compiler_params=pltpu.CompilerParams(
            dimension_semantics=("arbitrary",)),
    )(a2d, token_weight, position_weight)
    return out2d.reshape(batch, seq, N)
